# baseline (speedup 1.0000x reference)
import functools

import jax
import jax.numpy as jnp
from jax import lax
from jax.experimental import pallas as pl
from jax.experimental.pallas import tpu as pltpu
from jax.experimental.pallas import tpu_sc as plsc

_CHUNK = 64


@functools.lru_cache(maxsize=None)
def _make_gather(vocab: int, dim: int, batch: int):
    info = plsc.get_sparse_core_info()
    nw = info.num_cores * info.num_subcores
    b_per_w = batch // nw
    assert batch % (nw * _CHUNK) == 0
    n_chunks = b_per_w // _CHUNK
    main = (dim // 128) * 128
    tail = dim - main
    mesh = plsc.VectorSubcoreMesh(core_axis_name="c", subcore_axis_name="s")

    @functools.partial(
        pl.kernel,
        mesh=mesh,
        out_type=(jax.ShapeDtypeStruct((batch, main), jnp.float32),
                  jax.ShapeDtypeStruct((batch, 128), jnp.float32)),
        scratch_types=[
            pltpu.VMEM((b_per_w,), jnp.int32),
            pltpu.VMEM((_CHUNK, main), jnp.float32),
            pltpu.VMEM((_CHUNK, main), jnp.float32),
            pltpu.VMEM((_CHUNK, 128), jnp.float32),
            pltpu.VMEM((_CHUNK, 128), jnp.float32),
            pltpu.SemaphoreType.DMA,
            pltpu.SemaphoreType.DMA,
            pltpu.SemaphoreType.DMA,
            pltpu.SemaphoreType.DMA,
            pltpu.SemaphoreType.DMA,
            pltpu.SemaphoreType.DMA,
        ],
    )
    def gather(idx_hbm, table_hbm, tail_hbm, outm_hbm, outt_hbm, idx_v,
               main0, main1, tail0, tail1,
               gsem0, gsem1, tsem0, tsem1, ssem0, ssem1):
        wid = lax.axis_index("s") * info.num_cores + lax.axis_index("c")
        base = wid * b_per_w
        pltpu.sync_copy(idx_hbm.at[pl.ds(base, b_per_w)], idx_v)

        mains = (main0, main1)
        tails = (tail0, tail1)
        gsems = (gsem0, gsem1)
        tsems = (tsem0, tsem1)
        ssems = (ssem0, ssem1)
        gcopy = [None, None]
        tcopy = [None, None]
        scopy = [None, None, None, None]

        def start(i, b):
            ids = idx_v.at[pl.ds(i * _CHUNK, _CHUNK)]
            gcopy[b] = pltpu.async_copy(
                table_hbm.at[ids, pl.ds(0, main)], mains[b], gsems[b])
            tcopy[b] = pltpu.async_copy(tail_hbm.at[ids], tails[b], tsems[b])

        start(0, 0)
        for i in range(n_chunks):
            b = i & 1
            gcopy[b].wait()
            tcopy[b].wait()
            if i + 1 < n_chunks:
                start(i + 1, b ^ 1)
            if scopy[2 * b] is not None:
                scopy[2 * b].wait()
                scopy[2 * b + 1].wait()
            rows = pl.ds(base + i * _CHUNK, _CHUNK)
            scopy[2 * b] = pltpu.async_copy(
                mains[b], outm_hbm.at[rows], ssems[b])
            scopy[2 * b + 1] = pltpu.async_copy(
                tails[b], outt_hbm.at[rows], ssems[b])
        for c in scopy:
            if c is not None:
                c.wait()

    return gather


def kernel(news_ids, table):
    batch = news_ids.shape[0]
    vocab, dim = table.shape
    idx32 = news_ids.astype(jnp.int32)
    main = (dim // 128) * 128
    main_tbl = table[:, :main]
    tail_tbl = jnp.pad(table[:, main:], ((0, 0), (0, 128 - (dim - main))))
    out_main, out_tail = _make_gather(vocab, dim, batch)(
        idx32, main_tbl, tail_tbl)
    return jnp.concatenate([out_main, out_tail[:, :dim - main]], axis=1)


# speedup vs baseline: 1.0764x; 1.0764x over previous
import functools

import jax
import jax.numpy as jnp
from jax import lax
from jax.experimental import pallas as pl
from jax.experimental.pallas import tpu as pltpu
from jax.experimental.pallas import tpu_sc as plsc

_CHUNK = 64


@functools.lru_cache(maxsize=None)
def _make_gather(vocab: int, dim: int, batch: int):
    info = plsc.get_sparse_core_info()
    nw = info.num_cores * info.num_subcores
    b_per_w = batch // nw
    assert batch % (nw * _CHUNK) == 0
    n_chunks = b_per_w // _CHUNK
    main = (dim // 128) * 128
    tail = dim - main
    mesh = plsc.VectorSubcoreMesh(core_axis_name="c", subcore_axis_name="s")

    @functools.partial(
        pl.kernel,
        mesh=mesh,
        out_type=(jax.ShapeDtypeStruct((batch, main), jnp.float32),
                  jax.ShapeDtypeStruct((batch, 128), jnp.float32)),
        scratch_types=[
            pltpu.VMEM((b_per_w,), jnp.int32),
            pltpu.VMEM((_CHUNK, main), jnp.float32),
            pltpu.VMEM((_CHUNK, main), jnp.float32),
            pltpu.VMEM((_CHUNK, 128), jnp.float32),
            pltpu.VMEM((_CHUNK, 128), jnp.float32),
            pltpu.SemaphoreType.DMA,
            pltpu.SemaphoreType.DMA,
            pltpu.SemaphoreType.DMA,
            pltpu.SemaphoreType.DMA,
            pltpu.SemaphoreType.DMA,
            pltpu.SemaphoreType.DMA,
        ],
    )
    def gather(idx_hbm, table_hbm, tail_hbm, outm_hbm, outt_hbm, idx_v,
               main0, main1, tail0, tail1,
               gsem0, gsem1, tsem0, tsem1, ssem0, ssem1):
        wid = lax.axis_index("s") * info.num_cores + lax.axis_index("c")
        base = wid * b_per_w
        pltpu.sync_copy(idx_hbm.at[pl.ds(base, b_per_w)], idx_v)

        mains = (main0, main1)
        tails = (tail0, tail1)
        gsems = (gsem0, gsem1)
        tsems = (tsem0, tsem1)
        ssems = (ssem0, ssem1)
        gcopy = [None, None]
        tcopy = [None, None]
        scopy = [None, None, None, None]

        def start(i, b):
            ids = idx_v.at[pl.ds(i * _CHUNK, _CHUNK)]
            gcopy[b] = pltpu.async_copy(
                table_hbm.at[ids, pl.ds(0, main)], mains[b], gsems[b])
            tcopy[b] = pltpu.async_copy(tail_hbm.at[ids], tails[b], tsems[b])

        start(0, 0)
        for i in range(n_chunks):
            b = i & 1
            gcopy[b].wait()
            tcopy[b].wait()
            if i + 1 < n_chunks:
                start(i + 1, b ^ 1)
            if scopy[2 * b] is not None:
                scopy[2 * b].wait()
                scopy[2 * b + 1].wait()
            rows = pl.ds(base + i * _CHUNK, _CHUNK)
            scopy[2 * b] = pltpu.async_copy(
                mains[b], outm_hbm.at[rows], ssems[b])
            scopy[2 * b + 1] = pltpu.async_copy(
                tails[b], outt_hbm.at[rows], ssems[b])
        for c in scopy:
            if c is not None:
                c.wait()

    return gather


def kernel(news_ids, table):
    batch = news_ids.shape[0]
    vocab, dim = table.shape
    idx32 = news_ids.astype(jnp.int32)
    main = (dim // 128) * 128
    tail_tbl = jnp.pad(table[:, main:], ((0, 0), (0, 128 - (dim - main))))
    out_main, out_tail = _make_gather(vocab, dim, batch)(idx32, table, tail_tbl)
    return jnp.concatenate(
        [out_main.T, out_tail.T[:dim - main]], axis=0).T


# speedup vs baseline: 1.1425x; 1.0614x over previous
import functools

import jax
import jax.numpy as jnp
from jax import lax
from jax.experimental import pallas as pl
from jax.experimental.pallas import tpu as pltpu
from jax.experimental.pallas import tpu_sc as plsc

_CHUNK = 64


@functools.lru_cache(maxsize=None)
def _make_gather(vocab: int, dim: int, batch: int):
    info = plsc.get_sparse_core_info()
    nw = info.num_cores * info.num_subcores
    b_per_w = batch // nw
    assert batch % (nw * _CHUNK) == 0
    n_chunks = b_per_w // _CHUNK
    main = (dim // 128) * 128
    tail = dim - main
    mesh = plsc.VectorSubcoreMesh(core_axis_name="c", subcore_axis_name="s")

    @functools.partial(
        pl.kernel,
        mesh=mesh,
        out_type=jax.ShapeDtypeStruct((batch, main + 128), jnp.float32),
        scratch_types=[
            pltpu.VMEM((b_per_w,), jnp.int32),
            pltpu.VMEM((_CHUNK, main), jnp.float32),
            pltpu.VMEM((_CHUNK, main), jnp.float32),
            pltpu.VMEM((_CHUNK, 128), jnp.float32),
            pltpu.VMEM((_CHUNK, 128), jnp.float32),
            pltpu.SemaphoreType.DMA,
            pltpu.SemaphoreType.DMA,
            pltpu.SemaphoreType.DMA,
            pltpu.SemaphoreType.DMA,
            pltpu.SemaphoreType.DMA,
            pltpu.SemaphoreType.DMA,
        ],
    )
    def gather(idx_hbm, table_hbm, tail_hbm, out_hbm, idx_v,
               main0, main1, tail0, tail1,
               gsem0, gsem1, tsem0, tsem1, ssem0, ssem1):
        wid = lax.axis_index("s") * info.num_cores + lax.axis_index("c")
        base = wid * b_per_w
        pltpu.sync_copy(idx_hbm.at[pl.ds(base, b_per_w)], idx_v)

        mains = (main0, main1)
        tails = (tail0, tail1)
        gsems = (gsem0, gsem1)
        tsems = (tsem0, tsem1)
        ssems = (ssem0, ssem1)
        gcopy = [None, None]
        tcopy = [None, None]
        scopy = [None, None, None, None]

        def start(i, b):
            ids = idx_v.at[pl.ds(i * _CHUNK, _CHUNK)]
            gcopy[b] = pltpu.async_copy(
                table_hbm.at[ids, pl.ds(0, main)], mains[b], gsems[b])
            tcopy[b] = pltpu.async_copy(tail_hbm.at[ids], tails[b], tsems[b])

        start(0, 0)
        for i in range(n_chunks):
            b = i & 1
            gcopy[b].wait()
            tcopy[b].wait()
            if i + 1 < n_chunks:
                start(i + 1, b ^ 1)
            if scopy[2 * b] is not None:
                scopy[2 * b].wait()
                scopy[2 * b + 1].wait()
            rows = pl.ds(base + i * _CHUNK, _CHUNK)
            scopy[2 * b] = pltpu.async_copy(
                mains[b], out_hbm.at[rows, pl.ds(0, main)], ssems[b])
            scopy[2 * b + 1] = pltpu.async_copy(
                tails[b], out_hbm.at[rows, pl.ds(main, 128)], ssems[b])
        for c in scopy:
            if c is not None:
                c.wait()

    return gather


def kernel(news_ids, table):
    batch = news_ids.shape[0]
    vocab, dim = table.shape
    idx32 = news_ids.astype(jnp.int32)
    main = (dim // 128) * 128
    tail_tbl = jnp.pad(table[:, main:], ((0, 0), (0, 128 - (dim - main))))
    out_wide = _make_gather(vocab, dim, batch)(idx32, table, tail_tbl)
    return out_wide[:, :dim]
